# Initial kernel scaffold; baseline (speedup 1.0000x reference)
import functools

import jax
import jax.numpy as jnp
from jax import lax
from jax.experimental import pallas as pl
from jax.experimental.pallas import tpu as pltpu
from jax.experimental.pallas import tpu_sc as plsc

B = 16384
D = 16
N1, N2, NCAT = 200, 50, 20
CB = 16
G1, G2, GC = CB * N1, CB * N2, CB * NCAT


def _sc_pool(idx1, idx2, idxc, E1, E2, EC):
  info = plsc.get_sparse_core_info()
  nw = info.num_cores * info.num_subcores
  rows_w = B // nw
  n_chunks = rows_w // CB

  mesh = plsc.VectorSubcoreMesh(core_axis_name="c", subcore_axis_name="s")

  @functools.partial(
      pl.kernel,
      out_type=jax.ShapeDtypeStruct((B * 48,), jnp.float32),
      mesh=mesh,
      scratch_types=[
          pltpu.VMEM((G1,), jnp.int32),
          pltpu.VMEM((G2,), jnp.int32),
          pltpu.VMEM((GC,), jnp.int32),
          pltpu.VMEM((G1, D), jnp.float32),
          pltpu.VMEM((G2, D), jnp.float32),
          pltpu.VMEM((GC, D), jnp.float32),
          pltpu.VMEM((CB * 48,), jnp.float32),
          pltpu.SemaphoreType.DMA,
      ],
  )
  def k(idx1_h, idx2_h, idxc_h, e1_h, e2_h, ec_h, out_h,
        idx1_v, idx2_v, idxc_v, r1_v, r2_v, rc_v, out_v, sem):
    wid = lax.axis_index("s") * info.num_cores + lax.axis_index("c")
    col = lax.iota(jnp.int32, 16)

    @pl.loop(0, n_chunks)
    def _chunk(c):
      row0 = wid * rows_w + c * CB

      pltpu.sync_copy(idx1_h.at[pl.ds(row0 * N1, G1)], idx1_v)
      pltpu.sync_copy(idx2_h.at[pl.ds(row0 * N2, G2)], idx2_v)
      pltpu.sync_copy(idxc_h.at[pl.ds(row0 * NCAT, GC)], idxc_v)

      @pl.loop(0, G1 // 128)
      def _g1(g):
        pltpu.async_copy(e1_h.at[idx1_v.at[pl.ds(g * 128, 128)]],
                         r1_v.at[pl.ds(g * 128, 128)], sem)

      @pl.loop(0, G2 // 80)
      def _g2(g):
        pltpu.async_copy(e2_h.at[idx2_v.at[pl.ds(g * 80, 80)]],
                         r2_v.at[pl.ds(g * 80, 80)], sem)

      @pl.loop(0, GC // 80)
      def _gc(g):
        pltpu.async_copy(ec_h.at[idxc_v.at[pl.ds(g * 80, 80)]],
                         rc_v.at[pl.ds(g * 80, 80)], sem)

      pltpu.make_async_copy(e1_h.at[pl.ds(0, G1)], r1_v, sem).wait()
      pltpu.make_async_copy(e2_h.at[pl.ds(0, G2)], r2_v, sem).wait()
      pltpu.make_async_copy(ec_h.at[pl.ds(0, GC)], rc_v, sem).wait()

      @pl.loop(0, CB)
      def _acc(b):
        def row(rv, r):
          return plsc.load_gather(rv, [jnp.full((16,), r, jnp.int32), col])

        def s1(j, a):
          r = b * N1 + j * 8
          for t in range(8):
            a = a + row(r1_v, r + t)
          return a
        a1 = lax.fori_loop(0, N1 // 8, s1, jnp.zeros((16,), jnp.float32))

        def s2(j, a):
          r = b * N2 + j * 10
          for t in range(10):
            a = a + row(r2_v, r + t)
          return a
        a2 = lax.fori_loop(0, N2 // 10, s2, jnp.zeros((16,), jnp.float32))

        ac = jnp.zeros((16,), jnp.float32)
        for t in range(NCAT):
          ac = ac + row(rc_v, b * NCAT + t)

        out_v[pl.ds(b * 48, 16)] = a1 * (1.0 / N1)
        out_v[pl.ds(b * 48 + 16, 16)] = a2 * (1.0 / N2)
        out_v[pl.ds(b * 48 + 32, 16)] = ac * (1.0 / NCAT)

      pltpu.sync_copy(out_v, out_h.at[pl.ds(row0 * 48, CB * 48)])

  return k(idx1, idx2, idxc, E1, E2, EC)


def _mlp(pooled, numeric, w1a, w1n, b1, w2, b2):
  bm = 2048

  def body(x_ref, n_ref, w1a_ref, w1n_ref, b1_ref, w2_ref, b2_ref, o_ref):
    h = jnp.dot(x_ref[...], w1a_ref[...], preferred_element_type=jnp.float32)
    h = h + n_ref[...] * w1n_ref[...] + b1_ref[...]
    h = jnp.maximum(h, 0.0)
    o = jnp.dot(h, w2_ref[...], preferred_element_type=jnp.float32)
    o_ref[...] = jax.nn.sigmoid(o + b2_ref[...])

  return pl.pallas_call(
      body,
      grid=(B // bm,),
      in_specs=[
          pl.BlockSpec((bm, 48), lambda i: (i, 0)),
          pl.BlockSpec((bm, 1), lambda i: (i, 0)),
          pl.BlockSpec((48, 128), lambda i: (0, 0)),
          pl.BlockSpec((1, 128), lambda i: (0, 0)),
          pl.BlockSpec((1, 128), lambda i: (0, 0)),
          pl.BlockSpec((128, 1), lambda i: (0, 0)),
          pl.BlockSpec((1, 1), lambda i: (0, 0)),
      ],
      out_specs=pl.BlockSpec((bm, 1), lambda i: (i, 0)),
      out_shape=jax.ShapeDtypeStruct((B, 1), jnp.float32),
  )(pooled, numeric, w1a, w1n, b1, w2, b2)


def kernel(sentence_data_padded, company_data_padded, numeric_data,
           multi_class_cat_data, E1, E2, EC, W1, b1, W2, b2):
  idx1 = sentence_data_padded.reshape(-1).astype(jnp.int32)
  idx2 = company_data_padded.reshape(-1).astype(jnp.int32)
  idxc = multi_class_cat_data.reshape(-1).astype(jnp.int32)
  pooled = _sc_pool(idx1, idx2, idxc, E1, E2, EC).reshape(B, 48)
  return _mlp(pooled, numeric_data, W1[:48], W1[48:49],
              b1.reshape(1, 128), W2, b2.reshape(1, 1))


# speedup vs baseline: 10.3090x; 10.3090x over previous
import functools

import jax
import jax.numpy as jnp
from jax import lax
import numpy as np
from jax.experimental import pallas as pl
from jax.experimental.pallas import tpu as pltpu
from jax.experimental.pallas import tpu_sc as plsc

B = 16384
D = 16
N1, N2, NCAT = 200, 50, 20
CB = 16
G1, G2, GC = CB * N1, CB * N2, CB * NCAT
W1CH, W2CH, WCCH = 128, 80, 80


def _dst_rows(n_per_row, chunk, offset):
  i = np.arange(CB * n_per_row)
  base = (3 * (i // n_per_row) + offset).astype(np.int32).reshape(-1, chunk)
  s_off = (np.arange(16) * (CB * 3)).astype(np.int32)
  return np.ascontiguousarray(base[None] + s_off[:, None, None])


def _sc_pool(idx1, idx2, idxc, dix1, dix2, dixc, zeros, E1, E2, EC):
  info = plsc.get_sparse_core_info()
  nw = info.num_cores * info.num_subcores
  rows_w = B // nw
  n_chunks = rows_w // CB

  mesh = plsc.VectorSubcoreMesh(core_axis_name="c", subcore_axis_name="s")

  @functools.partial(
      pl.kernel,
      out_type=jax.ShapeDtypeStruct((B * 3, 16), jnp.float32),
      mesh=mesh,
      scratch_types=[
          pltpu.VMEM((G1,), jnp.int32),
          pltpu.VMEM((G2,), jnp.int32),
          pltpu.VMEM((GC,), jnp.int32),
          pltpu.VMEM((G1 // W1CH, W1CH), jnp.int32),
          pltpu.VMEM((G2 // W2CH, W2CH), jnp.int32),
          pltpu.VMEM((GC // WCCH, WCCH), jnp.int32),
          pltpu.VMEM((G1, D), jnp.float32),
          pltpu.VMEM((G2, D), jnp.float32),
          pltpu.VMEM((GC, D), jnp.float32),
          pltpu.VMEM_SHARED((16 * CB * 3, D), jnp.float32),
          pltpu.SemaphoreType.DMA,
      ],
      compiler_params=pltpu.CompilerParams(use_tc_tiling_on_sc=False),
  )
  def k(idx1_h, idx2_h, idxc_h, dix1_h, dix2_h, dixc_h, zero_h,
        e1_h, e2_h, ec_h, out_h,
        idx1_v, idx2_v, idxc_v, dix1_v, dix2_v, dixc_v,
        r1_v, r2_v, rc_v, acc_sh, sem):
    sid = lax.axis_index("s")
    wid = sid * info.num_cores + lax.axis_index("c")
    acc_v = acc_sh.at[pl.ds(sid * (CB * 3), CB * 3)]

    pltpu.sync_copy(dix1_h.at[sid], dix1_v)
    pltpu.sync_copy(dix2_h.at[sid], dix2_v)
    pltpu.sync_copy(dixc_h.at[sid], dixc_v)

    @pl.loop(0, n_chunks)
    def _chunk(c):
      row0 = wid * rows_w + c * CB

      pltpu.sync_copy(idx1_h.at[pl.ds(row0 * N1, G1)], idx1_v)
      pltpu.sync_copy(idx2_h.at[pl.ds(row0 * N2, G2)], idx2_v)
      pltpu.sync_copy(idxc_h.at[pl.ds(row0 * NCAT, GC)], idxc_v)
      pltpu.sync_copy(zero_h, acc_v)

      @pl.loop(0, G1 // W1CH)
      def _g1(g):
        pltpu.async_copy(e1_h.at[idx1_v.at[pl.ds(g * W1CH, W1CH)]],
                         r1_v.at[pl.ds(g * W1CH, W1CH)], sem)

      @pl.loop(0, G2 // W2CH)
      def _g2(g):
        pltpu.async_copy(e2_h.at[idx2_v.at[pl.ds(g * W2CH, W2CH)]],
                         r2_v.at[pl.ds(g * W2CH, W2CH)], sem)

      @pl.loop(0, GC // WCCH)
      def _gc(g):
        pltpu.async_copy(ec_h.at[idxc_v.at[pl.ds(g * WCCH, WCCH)]],
                         rc_v.at[pl.ds(g * WCCH, WCCH)], sem)

      pltpu.make_async_copy(e1_h.at[pl.ds(0, G1)], r1_v, sem).wait()
      pltpu.make_async_copy(e2_h.at[pl.ds(0, G2)], r2_v, sem).wait()
      pltpu.make_async_copy(ec_h.at[pl.ds(0, GC)], rc_v, sem).wait()

      @pl.loop(0, G1 // W1CH)
      def _s1(g):
        pltpu.async_copy(r1_v.at[pl.ds(g * W1CH, W1CH)],
                         acc_sh.at[dix1_v.at[g]], sem, add=True)

      @pl.loop(0, G2 // W2CH)
      def _s2(g):
        pltpu.async_copy(r2_v.at[pl.ds(g * W2CH, W2CH)],
                         acc_sh.at[dix2_v.at[g]], sem, add=True)

      @pl.loop(0, GC // WCCH)
      def _sc(g):
        pltpu.async_copy(rc_v.at[pl.ds(g * WCCH, WCCH)],
                         acc_sh.at[dixc_v.at[g]], sem, add=True)

      pltpu.make_async_copy(e1_h.at[pl.ds(0, G1)], r1_v, sem).wait()
      pltpu.make_async_copy(e2_h.at[pl.ds(0, G2)], r2_v, sem).wait()
      pltpu.make_async_copy(ec_h.at[pl.ds(0, GC)], rc_v, sem).wait()

      pltpu.sync_copy(acc_v, out_h.at[pl.ds(row0 * 3, CB * 3)])

  return k(idx1, idx2, idxc, dix1, dix2, dixc, zeros, E1, E2, EC)


def _mlp(pooled, numeric, scales, w1a, w1n, b1, w2, b2):
  bm = 2048

  def body(x_ref, n_ref, s_ref, w1a_ref, w1n_ref, b1_ref, w2_ref, b2_ref,
           o_ref):
    x = x_ref[...] * s_ref[...]
    h = jnp.dot(x, w1a_ref[...], preferred_element_type=jnp.float32)
    h = h + n_ref[...] * w1n_ref[...] + b1_ref[...]
    h = jnp.maximum(h, 0.0)
    o = jnp.dot(h, w2_ref[...], preferred_element_type=jnp.float32)
    o_ref[...] = jax.nn.sigmoid(o + b2_ref[...])

  return pl.pallas_call(
      body,
      grid=(B // bm,),
      in_specs=[
          pl.BlockSpec((bm, 48), lambda i: (i, 0)),
          pl.BlockSpec((bm, 1), lambda i: (i, 0)),
          pl.BlockSpec((1, 48), lambda i: (0, 0)),
          pl.BlockSpec((48, 128), lambda i: (0, 0)),
          pl.BlockSpec((1, 128), lambda i: (0, 0)),
          pl.BlockSpec((1, 128), lambda i: (0, 0)),
          pl.BlockSpec((128, 1), lambda i: (0, 0)),
          pl.BlockSpec((1, 1), lambda i: (0, 0)),
      ],
      out_specs=pl.BlockSpec((bm, 1), lambda i: (i, 0)),
      out_shape=jax.ShapeDtypeStruct((B, 1), jnp.float32),
  )(pooled, numeric, scales, w1a, w1n, b1, w2, b2)


_DIX1 = _dst_rows(N1, W1CH, 0)
_DIX2 = _dst_rows(N2, W2CH, 1)
_DIXC = _dst_rows(NCAT, WCCH, 2)
_SCALES = np.concatenate(
    [np.full(16, 1.0 / N1), np.full(16, 1.0 / N2),
     np.full(16, 1.0 / NCAT)]).astype(np.float32).reshape(1, 48)


def kernel(sentence_data_padded, company_data_padded, numeric_data,
           multi_class_cat_data, E1, E2, EC, W1, b1, W2, b2):
  idx1 = sentence_data_padded.astype(jnp.int32).reshape(-1)
  idx2 = company_data_padded.astype(jnp.int32).reshape(-1)
  idxc = multi_class_cat_data.astype(jnp.int32).reshape(-1)
  pooled = _sc_pool(idx1, idx2, idxc,
                    jnp.asarray(_DIX1), jnp.asarray(_DIX2), jnp.asarray(_DIXC),
                    jnp.zeros((CB * 3, D), jnp.float32),
                    E1, E2, EC).reshape(B, 48)
  return _mlp(pooled, numeric_data, jnp.asarray(_SCALES), W1[:48], W1[48:49],
              b1.reshape(1, 128), W2, b2.reshape(1, 1))


# speedup vs baseline: 12.0738x; 1.1712x over previous
import functools

import jax
import jax.numpy as jnp
from jax import lax
import numpy as np
from jax.experimental import pallas as pl
from jax.experimental.pallas import tpu as pltpu
from jax.experimental.pallas import tpu_sc as plsc

B = 16384
D = 16
N1, N2, NCAT = 200, 50, 20
CB = 8
G1, G2, GC = CB * N1, CB * N2, CB * NCAT
WCH = 80
ACC = CB * 3


def _dst_rows(n_per_row):
  i = np.arange(CB * n_per_row)
  off = {N1: 0, N2: 1, NCAT: 2}[n_per_row]
  return (3 * (i // n_per_row) + off).astype(np.int32).reshape(-1, WCH)


def _sc_pool(idx1, idx2, idxc, dix1, dix2, dixc, zeros, E1, E2, EC):
  info = plsc.get_sparse_core_info()
  nw = info.num_cores * info.num_subcores
  rows_w = B // nw
  n_chunks = rows_w // CB
  acc_w = rows_w * 3

  mesh = plsc.VectorSubcoreMesh(core_axis_name="c", subcore_axis_name="s")

  idx_shapes = [
      pltpu.VMEM((G1,), jnp.int32),
      pltpu.VMEM((G2,), jnp.int32),
      pltpu.VMEM((GC,), jnp.int32),
  ]
  row_shapes = [
      pltpu.VMEM((G1, D), jnp.float32),
      pltpu.VMEM((G2, D), jnp.float32),
      pltpu.VMEM((GC, D), jnp.float32),
  ]

  @functools.partial(
      pl.kernel,
      out_type=jax.ShapeDtypeStruct((B * 3, 16), jnp.float32),
      mesh=mesh,
      scratch_types=[
          idx_shapes, idx_shapes,
          row_shapes, row_shapes,
          [pltpu.VMEM((G1 // WCH, WCH), jnp.int32),
           pltpu.VMEM((G2 // WCH, WCH), jnp.int32),
           pltpu.VMEM((GC // WCH, WCH), jnp.int32)],
          pltpu.VMEM_SHARED((16 * 512 * 3, D), jnp.float32),
          [pltpu.SemaphoreType.DMA] * 2,
          [pltpu.SemaphoreType.DMA] * 2,
          [pltpu.SemaphoreType.DMA] * 2,
      ],
      compiler_params=pltpu.CompilerParams(use_tc_tiling_on_sc=False),
  )
  def k(idx1_h, idx2_h, idxc_h, dix1_h, dix2_h, dixc_h, zero_h,
        e1_h, e2_h, ec_h, out_h,
        idx_a, idx_b, rows_a, rows_b, dix_v, acc_sh, semg, sems, semi):
    sid = lax.axis_index("s")
    wid = sid * info.num_cores + lax.axis_index("c")
    accbase = sid * acc_w
    idx_sets, row_sets = (idx_a, idx_b), (rows_a, rows_b)
    idx_hs = (idx1_h, idx2_h, idxc_h)
    e_hs = (e1_h, e2_h, ec_h)
    gns = (G1, G2, GC)
    nps = (N1, N2, NCAT)

    def stage_idx(c, p, sem):
      row0 = wid * rows_w + c * CB
      for t in range(3):
        pltpu.async_copy(idx_hs[t].at[pl.ds(row0 * nps[t], gns[t])],
                         idx_sets[p][t], sem)

    def drain_idx(p, sem):
      for t in range(3):
        pltpu.make_async_copy(idx_hs[t].at[pl.ds(0, gns[t])],
                              idx_sets[p][t], sem).wait()

    def fire_gathers(p, sem):
      for t in range(3):
        @pl.loop(0, gns[t] // WCH)
        def _g(g, t=t):
          pltpu.async_copy(e_hs[t].at[idx_sets[p][t].at[pl.ds(g * WCH, WCH)]],
                           row_sets[p][t].at[pl.ds(g * WCH, WCH)], sem)

    def drain_rows(p, sem):
      for t in range(3):
        pltpu.make_async_copy(e_hs[t].at[pl.ds(0, gns[t])],
                              row_sets[p][t], sem).wait()

    def fire_scatters(c, p, sem):
      dst = acc_sh.at[pl.ds(accbase + c * ACC, ACC)]
      for t in range(3):
        @pl.loop(0, gns[t] // WCH)
        def _s(g, t=t):
          pltpu.async_copy(row_sets[p][t].at[pl.ds(g * WCH, WCH)],
                           dst.at[dix_v[t].at[g]], sem, add=True)

    pltpu.sync_copy(dix1_h, dix_v[0])
    pltpu.sync_copy(dix2_h, dix_v[1])
    pltpu.sync_copy(dixc_h, dix_v[2])
    pltpu.sync_copy(zero_h, acc_sh.at[pl.ds(accbase, acc_w)])
    stage_idx(0, 0, semi[0])
    stage_idx(1, 1, semi[1])
    drain_idx(0, semi[0])
    fire_gathers(0, semg[0])

    @pl.loop(0, n_chunks // 2)
    def _cc(cc):
      for p in range(2):
        c = cc * 2 + p
        q = 1 - p

        @pl.when(c + 1 < n_chunks)
        def _():
          drain_idx(q, semi[q])
          @pl.when(c >= 1)
          def _():
            drain_rows(q, sems[q])
          fire_gathers(q, semg[q])

        drain_rows(p, semg[p])
        fire_scatters(c, p, sems[p])

        @pl.when(c + 2 < n_chunks)
        def _():
          stage_idx(c + 2, p, semi[p])

    drain_rows(0, sems[0])
    drain_rows(1, sems[1])
    pltpu.sync_copy(acc_sh.at[pl.ds(accbase, acc_w)],
                    out_h.at[pl.ds(wid * acc_w, acc_w)])

  return k(idx1, idx2, idxc, dix1, dix2, dixc, zeros, E1, E2, EC)


def _mlp(pooled, numeric, scales, w1a, w1n, b1, w2, b2):
  bm = 2048

  def body(x_ref, n_ref, s_ref, w1a_ref, w1n_ref, b1_ref, w2_ref, b2_ref,
           o_ref):
    x = x_ref[...] * s_ref[...]
    h = jnp.dot(x, w1a_ref[...], preferred_element_type=jnp.float32)
    h = h + n_ref[...] * w1n_ref[...] + b1_ref[...]
    h = jnp.maximum(h, 0.0)
    o = jnp.dot(h, w2_ref[...], preferred_element_type=jnp.float32)
    o_ref[...] = jax.nn.sigmoid(o + b2_ref[...])

  return pl.pallas_call(
      body,
      grid=(B // bm,),
      in_specs=[
          pl.BlockSpec((bm, 48), lambda i: (i, 0)),
          pl.BlockSpec((bm, 1), lambda i: (i, 0)),
          pl.BlockSpec((1, 48), lambda i: (0, 0)),
          pl.BlockSpec((48, 128), lambda i: (0, 0)),
          pl.BlockSpec((1, 128), lambda i: (0, 0)),
          pl.BlockSpec((1, 128), lambda i: (0, 0)),
          pl.BlockSpec((128, 1), lambda i: (0, 0)),
          pl.BlockSpec((1, 1), lambda i: (0, 0)),
      ],
      out_specs=pl.BlockSpec((bm, 1), lambda i: (i, 0)),
      out_shape=jax.ShapeDtypeStruct((B, 1), jnp.float32),
  )(pooled, numeric, scales, w1a, w1n, b1, w2, b2)


_DIX1 = _dst_rows(N1)
_DIX2 = _dst_rows(N2)
_DIXC = _dst_rows(NCAT)
_SCALES = np.concatenate(
    [np.full(16, 1.0 / N1), np.full(16, 1.0 / N2),
     np.full(16, 1.0 / NCAT)]).astype(np.float32).reshape(1, 48)


def kernel(sentence_data_padded, company_data_padded, numeric_data,
           multi_class_cat_data, E1, E2, EC, W1, b1, W2, b2):
  idx1 = sentence_data_padded.astype(jnp.int32).reshape(-1)
  idx2 = company_data_padded.astype(jnp.int32).reshape(-1)
  idxc = multi_class_cat_data.astype(jnp.int32).reshape(-1)
  pooled = _sc_pool(idx1, idx2, idxc,
                    jnp.asarray(_DIX1), jnp.asarray(_DIX2), jnp.asarray(_DIXC),
                    jnp.zeros((512 * 3, D), jnp.float32),
                    E1, E2, EC).reshape(B, 48)
  return _mlp(pooled, numeric_data, jnp.asarray(_SCALES), W1[:48], W1[48:49],
              b1.reshape(1, 128), W2, b2.reshape(1, 1))


# speedup vs baseline: 12.1677x; 1.0078x over previous
import functools

import jax
import jax.numpy as jnp
from jax import lax
import numpy as np
from jax.experimental import pallas as pl
from jax.experimental.pallas import tpu as pltpu
from jax.experimental.pallas import tpu_sc as plsc

B = 16384
D = 16
N1, N2, NCAT = 200, 50, 20
P2, PC = 56, 24
CB = 8
ACC = CB * 3
S1A, S1B = 104, 96


def _sc_pool(idx1, idx2, idxc, dix1a, dix1b, dix2, dixc, zeros, E1, E2, EC):
  info = plsc.get_sparse_core_info()
  nw = info.num_cores * info.num_subcores
  rows_w = B // nw
  n_chunks = rows_w // CB
  acc_w = rows_w * 3

  mesh = plsc.VectorSubcoreMesh(core_axis_name="c", subcore_axis_name="s")

  idx_shapes = [
      pltpu.VMEM((CB, N1), jnp.int32),
      pltpu.VMEM((CB, N2), jnp.int32),
      pltpu.VMEM((CB, NCAT), jnp.int32),
  ]
  row_shapes = [
      pltpu.VMEM((CB * N1, D), jnp.float32),
      pltpu.VMEM((CB * P2, D), jnp.float32),
      pltpu.VMEM((CB * PC, D), jnp.float32),
  ]

  @functools.partial(
      pl.kernel,
      out_type=jax.ShapeDtypeStruct((B * 3, 16), jnp.float32),
      mesh=mesh,
      scratch_types=[
          idx_shapes, idx_shapes,
          row_shapes, row_shapes,
          [pltpu.VMEM((CB, S1A), jnp.int32),
           pltpu.VMEM((CB, S1B), jnp.int32),
           pltpu.VMEM((CB, N2), jnp.int32),
           pltpu.VMEM((CB, NCAT), jnp.int32)],
          pltpu.VMEM_SHARED((16 * 512 * 3, D), jnp.float32),
          [pltpu.SemaphoreType.DMA] * 2,
          [pltpu.SemaphoreType.DMA] * 2,
          [pltpu.SemaphoreType.DMA] * 2,
      ],
      compiler_params=pltpu.CompilerParams(use_tc_tiling_on_sc=False),
  )
  def k(idx1_h, idx2_h, idxc_h, dix1a_h, dix1b_h, dix2_h, dixc_h, zero_h,
        e1_h, e2_h, ec_h, out_h,
        idx_a, idx_b, rows_a, rows_b, dix_v, acc_sh, semg, sems, semi):
    sid = lax.axis_index("s")
    wid = sid * info.num_cores + lax.axis_index("c")
    accbase = sid * acc_w
    idx_sets, row_sets = (idx_a, idx_b), (rows_a, rows_b)
    idx_hs = (idx1_h, idx2_h, idxc_h)

    def stage_idx(c, p, sem):
      row0 = wid * rows_w + c * CB
      for t in range(3):
        pltpu.async_copy(idx_hs[t].at[pl.ds(row0, CB)], idx_sets[p][t], sem)

    def drain_idx(p, sem):
      for t in range(3):
        pltpu.make_async_copy(idx_hs[t].at[pl.ds(0, CB)],
                              idx_sets[p][t], sem).wait()

    def _streams(p):
      i1, i2, ic = idx_sets[p]
      r1, r2, rc = row_sets[p]
      out = []
      for b in range(CB):
        out.append((i1.at[b].at[pl.ds(0, S1A)],
                    r1.at[pl.ds(b * N1, S1A)], dix_v[0].at[b], e1_h))
        out.append((i1.at[b].at[pl.ds(S1A, S1B)],
                    r1.at[pl.ds(b * N1 + S1A, S1B)], dix_v[1].at[b], e1_h))
        out.append((i2.at[b], r2.at[pl.ds(b * P2, N2)], dix_v[2].at[b], e2_h))
        out.append((ic.at[b], rc.at[pl.ds(b * PC, NCAT)], dix_v[3].at[b], ec_h))
      return out

    def fire_gathers(p, sem):
      for iref, rslc, _, tab in _streams(p):
        pltpu.async_copy(tab.at[iref], rslc, sem)

    def drain_rows(p, sem):
      r1, r2, rc = row_sets[p]
      pltpu.make_async_copy(e1_h.at[pl.ds(0, CB * N1)], r1, sem).wait()
      for b in range(CB):
        pltpu.make_async_copy(e2_h.at[pl.ds(0, N2)],
                              r2.at[pl.ds(b * P2, N2)], sem).wait()
        pltpu.make_async_copy(ec_h.at[pl.ds(0, NCAT)],
                              rc.at[pl.ds(b * PC, NCAT)], sem).wait()

    def fire_scatters(c, p, sem):
      dst = acc_sh.at[pl.ds(accbase + c * ACC, ACC)]
      for _, rslc, dref, _ in _streams(p):
        pltpu.async_copy(rslc, dst.at[dref], sem, add=True)

    pltpu.sync_copy(dix1a_h, dix_v[0])
    pltpu.sync_copy(dix1b_h, dix_v[1])
    pltpu.sync_copy(dix2_h, dix_v[2])
    pltpu.sync_copy(dixc_h, dix_v[3])
    pltpu.sync_copy(zero_h, acc_sh.at[pl.ds(accbase, acc_w)])
    stage_idx(0, 0, semi[0])
    stage_idx(1, 1, semi[1])
    drain_idx(0, semi[0])
    fire_gathers(0, semg[0])

    @pl.loop(0, n_chunks // 2)
    def _cc(cc):
      for p in range(2):
        c = cc * 2 + p
        q = 1 - p

        @pl.when(c + 1 < n_chunks)
        def _():
          drain_idx(q, semi[q])
          @pl.when(c >= 1)
          def _():
            drain_rows(q, sems[q])
          fire_gathers(q, semg[q])

        drain_rows(p, semg[p])
        fire_scatters(c, p, sems[p])

        @pl.when(c + 2 < n_chunks)
        def _():
          stage_idx(c + 2, p, semi[p])

    drain_rows(0, sems[0])
    drain_rows(1, sems[1])
    pltpu.sync_copy(acc_sh.at[pl.ds(accbase, acc_w)],
                    out_h.at[pl.ds(wid * acc_w, acc_w)])

  return k(idx1, idx2, idxc, dix1a, dix1b, dix2, dixc, zeros, E1, E2, EC)


def _mlp(pooled, numeric, scales, w1a, w1n, b1, w2, b2):
  bm = 2048

  def body(x_ref, n_ref, s_ref, w1a_ref, w1n_ref, b1_ref, w2_ref, b2_ref,
           o_ref):
    x = x_ref[...] * s_ref[...]
    h = jnp.dot(x, w1a_ref[...], preferred_element_type=jnp.float32)
    h = h + n_ref[...] * w1n_ref[...] + b1_ref[...]
    h = jnp.maximum(h, 0.0)
    o = jnp.dot(h, w2_ref[...], preferred_element_type=jnp.float32)
    o_ref[...] = jax.nn.sigmoid(o + b2_ref[...])

  return pl.pallas_call(
      body,
      grid=(B // bm,),
      in_specs=[
          pl.BlockSpec((bm, 48), lambda i: (i, 0)),
          pl.BlockSpec((bm, 1), lambda i: (i, 0)),
          pl.BlockSpec((1, 48), lambda i: (0, 0)),
          pl.BlockSpec((48, 128), lambda i: (0, 0)),
          pl.BlockSpec((1, 128), lambda i: (0, 0)),
          pl.BlockSpec((1, 128), lambda i: (0, 0)),
          pl.BlockSpec((128, 1), lambda i: (0, 0)),
          pl.BlockSpec((1, 1), lambda i: (0, 0)),
      ],
      out_specs=pl.BlockSpec((bm, 1), lambda i: (i, 0)),
      out_shape=jax.ShapeDtypeStruct((B, 1), jnp.float32),
  )(pooled, numeric, scales, w1a, w1n, b1, w2, b2)


def _dix(width, off):
  b = np.arange(CB, dtype=np.int32) * 3 + off
  return np.repeat(b[:, None], width, axis=1)


_DIX1A = _dix(S1A, 0)
_DIX1B = _dix(S1B, 0)
_DIX2 = _dix(N2, 1)
_DIXC = _dix(NCAT, 2)
_SCALES = np.concatenate(
    [np.full(16, 1.0 / N1), np.full(16, 1.0 / N2),
     np.full(16, 1.0 / NCAT)]).astype(np.float32).reshape(1, 48)


def kernel(sentence_data_padded, company_data_padded, numeric_data,
           multi_class_cat_data, E1, E2, EC, W1, b1, W2, b2):
  idx1 = sentence_data_padded.astype(jnp.int32)
  idx2 = company_data_padded.astype(jnp.int32)
  idxc = multi_class_cat_data.astype(jnp.int32)
  pooled = _sc_pool(idx1, idx2, idxc,
                    jnp.asarray(_DIX1A), jnp.asarray(_DIX1B),
                    jnp.asarray(_DIX2), jnp.asarray(_DIXC),
                    jnp.zeros((512 * 3, D), jnp.float32),
                    E1, E2, EC).reshape(B, 48)
  return _mlp(pooled, numeric_data, jnp.asarray(_SCALES), W1[:48], W1[48:49],
              b1.reshape(1, 128), W2, b2.reshape(1, 1))
